# baseline (speedup 1.0000x reference)
import functools

import jax
import jax.numpy as jnp
from jax import lax
from jax.experimental import pallas as pl
from jax.experimental.pallas import tpu as pltpu
from jax.experimental.pallas import tpu_sc as plsc

N_NODES = 100000
N_EDGES = 1600000
NUM_GRAPHS = 2048
ATOM_VOCAB = 100
EDGE_VOCAB = 8
TIME_STEPS = 1000
MAX_NODE_COUNT = 512
D_NODE = 128
D_EDGE = 32

NC = 2
NS = 16
NW = NC * NS
G_PER_W = NUM_GRAPHS // NW

CHUNK = 128
N_FULL_NODE_CHUNKS = N_NODES // CHUNK
NODE_TAIL = N_NODES - N_FULL_NODE_CHUNKS * CHUNK
N_EDGE_CHUNKS = N_EDGES // CHUNK

_mesh = plsc.VectorSubcoreMesh(core_axis_name="c", subcore_axis_name="s")
_params = pltpu.CompilerParams(needs_layout_passes=False,
                               use_tc_tiling_on_sc=False)


def _wid():
    return lax.axis_index("s") * NC + lax.axis_index("c")


@functools.partial(
    pl.kernel,
    out_type=jax.ShapeDtypeStruct((NUM_GRAPHS, 2 * D_NODE), jnp.float32),
    mesh=_mesh,
    compiler_params=_params,
    scratch_types=[
        pltpu.VMEM((N_NODES,), jnp.int32),
        pltpu.VMEM((G_PER_W + 16,), jnp.int32),
        pltpu.VMEM((G_PER_W,), jnp.int32),
        pltpu.VMEM((G_PER_W,), jnp.int32),
        pltpu.VMEM((G_PER_W, D_NODE), jnp.float32),
        pltpu.VMEM((G_PER_W, D_NODE), jnp.float32),
        pltpu.SemaphoreType.DMA,
    ],
)
def _combined_kernel(batch_hbm, t_hbm, nct_hbm, tt_hbm, out_hbm,
                     batch_v, starts_v, nc_idx_v, t_idx_v, nc_rows_v,
                     t_rows_v, sem):
    wid = _wid()
    g0 = wid * G_PER_W
    pltpu.sync_copy(batch_hbm, batch_v)
    lane = lax.iota(jnp.int32, 16)
    for j in range(G_PER_W // 16 + 1):
        g_vec = g0 + j * 16 + lane

        def body(_, carry, g_vec=g_vec):
            lo, hi = carry
            active = lo < hi
            mid = lax.shift_right_logical(lo + hi, 1)
            vals = plsc.load_gather(
                batch_v, [jnp.minimum(mid, N_NODES - 1)])
            go = jnp.logical_and(active, vals < g_vec)
            lo = jnp.where(go, mid + 1, lo)
            hi = jnp.where(jnp.logical_and(active, jnp.logical_not(go)),
                           mid, hi)
            return lo, hi

        lo = jnp.zeros((16,), jnp.int32)
        hi = jnp.full((16,), N_NODES, jnp.int32)
        lo, hi = lax.fori_loop(0, 17, body, (lo, hi))
        starts_v[pl.ds(j * 16, 16)] = lo
    for j in range(G_PER_W // 16):
        s0 = starts_v[pl.ds(j * 16, 16)]
        s1 = starts_v[pl.ds(j * 16 + 1, 16)]
        n = s1 - s0
        nc_idx_v[pl.ds(j * 16, 16)] = jnp.minimum(n, MAX_NODE_COUNT - 1)
    pltpu.sync_copy(t_hbm.at[pl.ds(g0, G_PER_W)], t_idx_v)
    pltpu.async_copy(nct_hbm.at[nc_idx_v], nc_rows_v, sem).wait()
    pltpu.async_copy(tt_hbm.at[t_idx_v], t_rows_v, sem).wait()
    pltpu.sync_copy(nc_rows_v,
                    out_hbm.at[pl.ds(g0, G_PER_W), pl.ds(0, D_NODE)])
    pltpu.sync_copy(t_rows_v,
                    out_hbm.at[pl.ds(g0, G_PER_W), pl.ds(D_NODE, D_NODE)])


@functools.partial(
    pl.kernel,
    out_type=jax.ShapeDtypeStruct((N_NODES, 3 * D_NODE), jnp.float32),
    mesh=_mesh,
    compiler_params=_params,
    scratch_types=[
        pltpu.VMEM((CHUNK,), jnp.int32),
        pltpu.VMEM((CHUNK,), jnp.int32),
        pltpu.VMEM((CHUNK, D_NODE), jnp.float32),
        pltpu.VMEM((CHUNK, 2 * D_NODE), jnp.float32),
        pltpu.SemaphoreType.DMA,
        pltpu.SemaphoreType.DMA,
    ],
)
def _h0_kernel(a_hbm, batch_hbm, atom_hbm, comb_hbm, out_hbm,
               a_idx, b_idx, atom_rows, comb_rows, sem_a, sem_b):
    wid = _wid()
    niter = (N_FULL_NODE_CHUNKS - wid + NW - 1) // NW

    def chunk_body(i, _):
        base = (wid + i * NW) * CHUNK
        pltpu.sync_copy(a_hbm.at[pl.ds(base, CHUNK)], a_idx)
        pltpu.sync_copy(batch_hbm.at[pl.ds(base, CHUNK)], b_idx)
        ca = pltpu.async_copy(atom_hbm.at[a_idx], atom_rows, sem_a)
        cb = pltpu.async_copy(comb_hbm.at[b_idx], comb_rows, sem_b)
        ca.wait()
        cb.wait()
        pltpu.sync_copy(atom_rows,
                        out_hbm.at[pl.ds(base, CHUNK), pl.ds(0, D_NODE)])
        pltpu.sync_copy(
            comb_rows,
            out_hbm.at[pl.ds(base, CHUNK), pl.ds(D_NODE, 2 * D_NODE)])
        return 0

    lax.fori_loop(0, niter, chunk_body, 0)

    @pl.when(wid == NW - 1)
    def _tail():
        base = N_FULL_NODE_CHUNKS * CHUNK
        pltpu.sync_copy(a_hbm.at[pl.ds(base, NODE_TAIL)],
                        a_idx.at[pl.ds(0, NODE_TAIL)])
        pltpu.sync_copy(batch_hbm.at[pl.ds(base, NODE_TAIL)],
                        b_idx.at[pl.ds(0, NODE_TAIL)])
        ca = pltpu.async_copy(atom_hbm.at[a_idx.at[pl.ds(0, NODE_TAIL)]],
                              atom_rows.at[pl.ds(0, NODE_TAIL)], sem_a)
        cb = pltpu.async_copy(comb_hbm.at[b_idx.at[pl.ds(0, NODE_TAIL)]],
                              comb_rows.at[pl.ds(0, NODE_TAIL)], sem_b)
        ca.wait()
        cb.wait()
        pltpu.sync_copy(
            atom_rows.at[pl.ds(0, NODE_TAIL)],
            out_hbm.at[pl.ds(base, NODE_TAIL), pl.ds(0, D_NODE)])
        pltpu.sync_copy(
            comb_rows.at[pl.ds(0, NODE_TAIL)],
            out_hbm.at[pl.ds(base, NODE_TAIL), pl.ds(D_NODE, 2 * D_NODE)])


@functools.partial(
    pl.kernel,
    out_type=jax.ShapeDtypeStruct((N_EDGES, D_EDGE), jnp.float32),
    mesh=_mesh,
    compiler_params=_params,
    scratch_types=[
        pltpu.VMEM((CHUNK,), jnp.int32),
        pltpu.VMEM((CHUNK, D_EDGE), jnp.float32),
        pltpu.SemaphoreType.DMA,
    ],
)
def _edge_kernel(e_hbm, et_hbm, out_hbm, e_idx, e_rows, sem):
    wid = _wid()
    niter = (N_EDGE_CHUNKS - wid + NW - 1) // NW

    def chunk_body(i, _):
        base = (wid + i * NW) * CHUNK
        pltpu.sync_copy(e_hbm.at[pl.ds(base, CHUNK)], e_idx)
        pltpu.async_copy(et_hbm.at[e_idx], e_rows, sem).wait()
        pltpu.sync_copy(e_rows, out_hbm.at[pl.ds(base, CHUNK)])
        return 0

    lax.fori_loop(0, niter, chunk_body, 0)


def kernel(a, c, e, edge_index, t, batch, atom_table, node_count_table,
           time_table, edge_table):
    del c
    a = a.astype(jnp.int32)
    e = e.astype(jnp.int32)
    t = t.astype(jnp.int32)
    batch = batch.astype(jnp.int32)
    combined = _combined_kernel(batch, t, node_count_table, time_table)
    h_0 = _h0_kernel(a, batch, atom_table, combined)
    e_embed = _edge_kernel(e, edge_table)
    return (h_0, (edge_index[0], edge_index[1]), e_embed)


# speedup vs baseline: 1.0062x; 1.0062x over previous
import functools

import jax
import jax.numpy as jnp
from jax import lax
from jax.experimental import pallas as pl
from jax.experimental.pallas import tpu as pltpu
from jax.experimental.pallas import tpu_sc as plsc

N_NODES = 100000
N_EDGES = 1600000
NUM_GRAPHS = 2048
ATOM_VOCAB = 100
EDGE_VOCAB = 8
TIME_STEPS = 1000
MAX_NODE_COUNT = 512
D_NODE = 128
D_EDGE = 32

NC = 2
NS = 16
NW = NC * NS
G_PER_W = NUM_GRAPHS // NW

CHUNK = 128
N_FULL_NODE_CHUNKS = N_NODES // CHUNK
NODE_TAIL = N_NODES - N_FULL_NODE_CHUNKS * CHUNK
N_EDGE_CHUNKS = N_EDGES // CHUNK

_mesh = plsc.VectorSubcoreMesh(core_axis_name="c", subcore_axis_name="s")
_params = pltpu.CompilerParams(needs_layout_passes=False,
                               use_tc_tiling_on_sc=False)


def _wid():
    return lax.axis_index("s") * NC + lax.axis_index("c")


@functools.partial(
    pl.kernel,
    out_type=jax.ShapeDtypeStruct((NUM_GRAPHS, 2 * D_NODE), jnp.float32),
    mesh=_mesh,
    compiler_params=_params,
    scratch_types=[
        pltpu.VMEM((N_NODES,), jnp.int32),
        pltpu.VMEM((G_PER_W + 16,), jnp.int32),
        pltpu.VMEM((G_PER_W,), jnp.int32),
        pltpu.VMEM((G_PER_W,), jnp.int32),
        pltpu.VMEM((G_PER_W, D_NODE), jnp.float32),
        pltpu.VMEM((G_PER_W, D_NODE), jnp.float32),
        pltpu.SemaphoreType.DMA,
    ],
)
def _combined_kernel(batch_hbm, t_hbm, nct_hbm, tt_hbm, out_hbm,
                     batch_v, starts_v, nc_idx_v, t_idx_v, nc_rows_v,
                     t_rows_v, sem):
    wid = _wid()
    g0 = wid * G_PER_W
    pltpu.sync_copy(batch_hbm, batch_v)
    lane = lax.iota(jnp.int32, 16)
    for j in range(G_PER_W // 16 + 1):
        g_vec = g0 + j * 16 + lane

        def body(_, carry, g_vec=g_vec):
            lo, hi = carry
            active = lo < hi
            mid = lax.shift_right_logical(lo + hi, 1)
            vals = plsc.load_gather(
                batch_v, [jnp.minimum(mid, N_NODES - 1)])
            go = jnp.logical_and(active, vals < g_vec)
            lo = jnp.where(go, mid + 1, lo)
            hi = jnp.where(jnp.logical_and(active, jnp.logical_not(go)),
                           mid, hi)
            return lo, hi

        lo = jnp.zeros((16,), jnp.int32)
        hi = jnp.full((16,), N_NODES, jnp.int32)
        lo, hi = lax.fori_loop(0, 17, body, (lo, hi))
        starts_v[pl.ds(j * 16, 16)] = lo
    for j in range(G_PER_W // 16):
        s0 = starts_v[pl.ds(j * 16, 16)]
        s1 = starts_v[pl.ds(j * 16 + 1, 16)]
        n = s1 - s0
        nc_idx_v[pl.ds(j * 16, 16)] = jnp.minimum(n, MAX_NODE_COUNT - 1)
    pltpu.sync_copy(t_hbm.at[pl.ds(g0, G_PER_W)], t_idx_v)
    pltpu.async_copy(nct_hbm.at[nc_idx_v], nc_rows_v, sem).wait()
    pltpu.async_copy(tt_hbm.at[t_idx_v], t_rows_v, sem).wait()
    pltpu.sync_copy(nc_rows_v,
                    out_hbm.at[pl.ds(g0, G_PER_W), pl.ds(0, D_NODE)])
    pltpu.sync_copy(t_rows_v,
                    out_hbm.at[pl.ds(g0, G_PER_W), pl.ds(D_NODE, D_NODE)])


@functools.partial(
    pl.kernel,
    out_type=jax.ShapeDtypeStruct((N_NODES, 3 * D_NODE), jnp.float32),
    mesh=_mesh,
    compiler_params=_params,
    scratch_types=[
        pltpu.VMEM((2, CHUNK), jnp.int32),
        pltpu.VMEM((2, CHUNK), jnp.int32),
        pltpu.VMEM((2, CHUNK, D_NODE), jnp.float32),
        pltpu.VMEM((2, CHUNK, 2 * D_NODE), jnp.float32),
        pltpu.SemaphoreType.DMA,
        pltpu.SemaphoreType.DMA,
        pltpu.SemaphoreType.DMA,
        pltpu.SemaphoreType.DMA,
        pltpu.SemaphoreType.DMA,
        pltpu.SemaphoreType.DMA,
    ],
)
def _h0_kernel(a_hbm, batch_hbm, atom_hbm, comb_hbm, out_hbm,
               a_idx, b_idx, atom2, comb2, si0, si1, sg0, sg1, sw0, sw1):
    wid = _wid()
    sems_i = (si0, si1)
    sems_g = (sg0, sg1)
    sems_w = (sw0, sw1)

    def idx_load(s, base, sem):
        pltpu.async_copy(a_hbm.at[pl.ds(base, CHUNK)], a_idx.at[s], sem)
        pltpu.async_copy(batch_hbm.at[pl.ds(base, CHUNK)], b_idx.at[s], sem)

    def idx_wait(s, base, sem):
        pltpu.make_async_copy(a_hbm.at[pl.ds(base, CHUNK)], a_idx.at[s],
                              sem).wait()
        pltpu.make_async_copy(batch_hbm.at[pl.ds(base, CHUNK)], b_idx.at[s],
                              sem).wait()

    def out_slices(base):
        return (out_hbm.at[pl.ds(base, CHUNK), pl.ds(0, D_NODE)],
                out_hbm.at[pl.ds(base, CHUNK), pl.ds(D_NODE, 2 * D_NODE)])

    for s in (0, 1):
        idx_load(s, (wid + s * NW) * CHUNK, sems_i[s])

    def pair_body(i, _):
        for s in (0, 1):
            k = 2 * i + s
            c = wid + k * NW
            base = c * CHUNK

            @pl.when(c < N_FULL_NODE_CHUNKS)
            def _(k=k, c=c, base=base, s=s):
                o_a, o_c = out_slices(base)
                @pl.when(k >= 2)
                def _():
                    pltpu.make_async_copy(atom2.at[s], o_a, sems_w[s]).wait()
                    pltpu.make_async_copy(comb2.at[s], o_c, sems_w[s]).wait()

                idx_wait(s, base, sems_i[s])
                ga = pltpu.async_copy(atom_hbm.at[a_idx.at[s]], atom2.at[s],
                                      sems_g[s])
                gc = pltpu.async_copy(comb_hbm.at[b_idx.at[s]], comb2.at[s],
                                      sems_g[s])
                ga.wait()
                gc.wait()

                @pl.when(c + 2 * NW < N_FULL_NODE_CHUNKS)
                def _():
                    idx_load(s, (c + 2 * NW) * CHUNK, sems_i[s])

                pltpu.async_copy(atom2.at[s], o_a, sems_w[s])
                pltpu.async_copy(comb2.at[s], o_c, sems_w[s])
        return 0

    max_pairs = (N_FULL_NODE_CHUNKS // NW + 2) // 2
    lax.fori_loop(0, max_pairs, pair_body, 0)

    for s in (0, 1):
        o_a, o_c = out_slices(wid * CHUNK)
        pltpu.make_async_copy(atom2.at[s], o_a, sems_w[s]).wait()
        pltpu.make_async_copy(comb2.at[s], o_c, sems_w[s]).wait()

    @pl.when(wid == NW - 1)
    def _tail():
        base = N_FULL_NODE_CHUNKS * CHUNK
        pltpu.sync_copy(a_hbm.at[pl.ds(base, NODE_TAIL)],
                        a_idx.at[0, pl.ds(0, NODE_TAIL)])
        pltpu.sync_copy(batch_hbm.at[pl.ds(base, NODE_TAIL)],
                        b_idx.at[0, pl.ds(0, NODE_TAIL)])
        ca = pltpu.async_copy(atom_hbm.at[a_idx.at[0, pl.ds(0, NODE_TAIL)]],
                              atom2.at[0, pl.ds(0, NODE_TAIL)], sg0)
        cb = pltpu.async_copy(comb_hbm.at[b_idx.at[0, pl.ds(0, NODE_TAIL)]],
                              comb2.at[0, pl.ds(0, NODE_TAIL)], sg1)
        ca.wait()
        cb.wait()
        pltpu.sync_copy(
            atom2.at[0, pl.ds(0, NODE_TAIL)],
            out_hbm.at[pl.ds(base, NODE_TAIL), pl.ds(0, D_NODE)])
        pltpu.sync_copy(
            comb2.at[0, pl.ds(0, NODE_TAIL)],
            out_hbm.at[pl.ds(base, NODE_TAIL), pl.ds(D_NODE, 2 * D_NODE)])


EDGES_PER_TILE = N_EDGES // NW
E_GROUP = 8 * CHUNK
E_NGROUPS = EDGES_PER_TILE // E_GROUP
E_TAIL = EDGES_PER_TILE - E_NGROUPS * E_GROUP
E_TAIL_FULL = E_TAIL // CHUNK
E_TAIL_REM = E_TAIL - E_TAIL_FULL * CHUNK


@functools.partial(
    pl.kernel,
    out_type=jax.ShapeDtypeStruct((N_EDGES, D_EDGE), jnp.float32),
    mesh=_mesh,
    compiler_params=_params,
    scratch_types=[
        pltpu.VMEM((2, E_GROUP), jnp.int32),
        pltpu.VMEM((2, E_GROUP, D_EDGE), jnp.float32),
        pltpu.SemaphoreType.DMA,
        pltpu.SemaphoreType.DMA,
        pltpu.SemaphoreType.DMA,
        pltpu.SemaphoreType.DMA,
        pltpu.SemaphoreType.DMA,
        pltpu.SemaphoreType.DMA,
    ],
)
def _edge_kernel(e_hbm, et_hbm, out_hbm, idx2, rows2,
                 si0, si1, sg0, sg1, sw0, sw1):
    wid = _wid()
    e0 = wid * EDGES_PER_TILE
    sems_i = (si0, si1)
    sems_g = (sg0, sg1)
    sems_w = (sw0, sw1)

    for s in (0, 1):
        pltpu.async_copy(e_hbm.at[pl.ds(e0 + s * E_GROUP, E_GROUP)],
                         idx2.at[s], sems_i[s])

    def pair_body(i, _):
        for s in (0, 1):
            g = 2 * i + s
            base = e0 + g * E_GROUP

            @pl.when(g >= 2)
            def _(s=s, base=base):
                pltpu.make_async_copy(rows2.at[s],
                                      out_hbm.at[pl.ds(base, E_GROUP)],
                                      sems_w[s]).wait()

            pltpu.make_async_copy(e_hbm.at[pl.ds(base, E_GROUP)],
                                  idx2.at[s], sems_i[s]).wait()
            for j in range(E_GROUP // CHUNK):
                pltpu.async_copy(
                    et_hbm.at[idx2.at[s, pl.ds(j * CHUNK, CHUNK)]],
                    rows2.at[s, pl.ds(j * CHUNK, CHUNK)], sems_g[s])
            for j in range(E_GROUP // CHUNK):
                pltpu.make_async_copy(
                    et_hbm.at[idx2.at[s, pl.ds(j * CHUNK, CHUNK)]],
                    rows2.at[s, pl.ds(j * CHUNK, CHUNK)], sems_g[s]).wait()

            @pl.when(g + 2 < E_NGROUPS)
            def _(s=s, base=base):
                pltpu.async_copy(e_hbm.at[pl.ds(base + 2 * E_GROUP, E_GROUP)],
                                 idx2.at[s], sems_i[s])

            pltpu.async_copy(rows2.at[s], out_hbm.at[pl.ds(base, E_GROUP)],
                             sems_w[s])
        return 0

    lax.fori_loop(0, E_NGROUPS // 2, pair_body, 0)

    for s in (0, 1):
        pltpu.make_async_copy(rows2.at[s], out_hbm.at[pl.ds(e0, E_GROUP)],
                              sems_w[s]).wait()

    tbase = e0 + E_NGROUPS * E_GROUP
    pltpu.sync_copy(e_hbm.at[pl.ds(tbase, E_TAIL)],
                    idx2.at[0, pl.ds(0, E_TAIL)])
    for j in range(E_TAIL_FULL):
        pltpu.async_copy(et_hbm.at[idx2.at[0, pl.ds(j * CHUNK, CHUNK)]],
                         rows2.at[0, pl.ds(j * CHUNK, CHUNK)], sg0)
    pltpu.async_copy(
        et_hbm.at[idx2.at[0, pl.ds(E_TAIL_FULL * CHUNK, E_TAIL_REM)]],
        rows2.at[0, pl.ds(E_TAIL_FULL * CHUNK, E_TAIL_REM)], sg0)
    for j in range(E_TAIL_FULL):
        pltpu.make_async_copy(
            et_hbm.at[idx2.at[0, pl.ds(j * CHUNK, CHUNK)]],
            rows2.at[0, pl.ds(j * CHUNK, CHUNK)], sg0).wait()
    pltpu.make_async_copy(
        et_hbm.at[idx2.at[0, pl.ds(E_TAIL_FULL * CHUNK, E_TAIL_REM)]],
        rows2.at[0, pl.ds(E_TAIL_FULL * CHUNK, E_TAIL_REM)], sg0).wait()
    pltpu.sync_copy(rows2.at[0, pl.ds(0, E_TAIL)],
                    out_hbm.at[pl.ds(tbase, E_TAIL)])


def kernel(a, c, e, edge_index, t, batch, atom_table, node_count_table,
           time_table, edge_table):
    del c
    a = a.astype(jnp.int32)
    e = e.astype(jnp.int32)
    t = t.astype(jnp.int32)
    batch = batch.astype(jnp.int32)
    combined = _combined_kernel(batch, t, node_count_table, time_table)
    h_0 = _h0_kernel(a, batch, atom_table, combined)
    e_embed = _edge_kernel(e, edge_table)
    return (h_0, (edge_index[0], edge_index[1]), e_embed)


# speedup vs baseline: 3.1104x; 3.0913x over previous
import functools

import jax
import jax.numpy as jnp
from jax import lax
from jax.experimental import pallas as pl
from jax.experimental.pallas import tpu as pltpu
from jax.experimental.pallas import tpu_sc as plsc

N_NODES = 100000
N_EDGES = 1600000
NUM_GRAPHS = 2048
ATOM_VOCAB = 100
EDGE_VOCAB = 8
TIME_STEPS = 1000
MAX_NODE_COUNT = 512
D_NODE = 128
D_EDGE = 32

NC = 2
NS = 16
NW = NC * NS
G_PER_W = NUM_GRAPHS // NW

CHUNK = 128
N_FULL_NODE_CHUNKS = N_NODES // CHUNK
NODE_TAIL = N_NODES - N_FULL_NODE_CHUNKS * CHUNK
N_EDGE_CHUNKS = N_EDGES // CHUNK

_mesh = plsc.VectorSubcoreMesh(core_axis_name="c", subcore_axis_name="s")
_params = pltpu.CompilerParams(needs_layout_passes=False,
                               use_tc_tiling_on_sc=False)


def _wid():
    return lax.axis_index("s") * NC + lax.axis_index("c")


@functools.partial(
    pl.kernel,
    out_type=jax.ShapeDtypeStruct((NUM_GRAPHS, 2 * D_NODE), jnp.float32),
    mesh=_mesh,
    compiler_params=_params,
    scratch_types=[
        pltpu.VMEM((N_NODES,), jnp.int32),
        pltpu.VMEM((G_PER_W + 16,), jnp.int32),
        pltpu.VMEM((G_PER_W,), jnp.int32),
        pltpu.VMEM((G_PER_W,), jnp.int32),
        pltpu.VMEM((G_PER_W, D_NODE), jnp.float32),
        pltpu.VMEM((G_PER_W, D_NODE), jnp.float32),
        pltpu.SemaphoreType.DMA,
    ],
)
def _combined_kernel(batch_hbm, t_hbm, nct_hbm, tt_hbm, out_hbm,
                     batch_v, starts_v, nc_idx_v, t_idx_v, nc_rows_v,
                     t_rows_v, sem):
    wid = _wid()
    g0 = wid * G_PER_W
    pltpu.sync_copy(batch_hbm, batch_v)
    lane = lax.iota(jnp.int32, 16)
    for j in range(G_PER_W // 16 + 1):
        g_vec = g0 + j * 16 + lane

        def body(_, carry, g_vec=g_vec):
            lo, hi = carry
            active = lo < hi
            mid = lax.shift_right_logical(lo + hi, 1)
            vals = plsc.load_gather(
                batch_v, [jnp.minimum(mid, N_NODES - 1)])
            go = jnp.logical_and(active, vals < g_vec)
            lo = jnp.where(go, mid + 1, lo)
            hi = jnp.where(jnp.logical_and(active, jnp.logical_not(go)),
                           mid, hi)
            return lo, hi

        lo = jnp.zeros((16,), jnp.int32)
        hi = jnp.full((16,), N_NODES, jnp.int32)
        lo, hi = lax.fori_loop(0, 17, body, (lo, hi))
        starts_v[pl.ds(j * 16, 16)] = lo
    for j in range(G_PER_W // 16):
        s0 = starts_v[pl.ds(j * 16, 16)]
        s1 = starts_v[pl.ds(j * 16 + 1, 16)]
        n = s1 - s0
        nc_idx_v[pl.ds(j * 16, 16)] = jnp.minimum(n, MAX_NODE_COUNT - 1)
    pltpu.sync_copy(t_hbm.at[pl.ds(g0, G_PER_W)], t_idx_v)
    pltpu.async_copy(nct_hbm.at[nc_idx_v], nc_rows_v, sem).wait()
    pltpu.async_copy(tt_hbm.at[t_idx_v], t_rows_v, sem).wait()
    pltpu.sync_copy(nc_rows_v,
                    out_hbm.at[pl.ds(g0, G_PER_W), pl.ds(0, D_NODE)])
    pltpu.sync_copy(t_rows_v,
                    out_hbm.at[pl.ds(g0, G_PER_W), pl.ds(D_NODE, D_NODE)])


@functools.partial(
    pl.kernel,
    out_type=jax.ShapeDtypeStruct((N_NODES, 3 * D_NODE), jnp.float32),
    mesh=_mesh,
    compiler_params=_params,
    scratch_types=[
        pltpu.VMEM((2, CHUNK), jnp.int32),
        pltpu.VMEM((2, CHUNK), jnp.int32),
        pltpu.VMEM((2, CHUNK, D_NODE), jnp.float32),
        pltpu.VMEM((2, CHUNK, 2 * D_NODE), jnp.float32),
        pltpu.SemaphoreType.DMA,
        pltpu.SemaphoreType.DMA,
        pltpu.SemaphoreType.DMA,
        pltpu.SemaphoreType.DMA,
        pltpu.SemaphoreType.DMA,
        pltpu.SemaphoreType.DMA,
    ],
)
def _h0_kernel(a_hbm, batch_hbm, atom_hbm, comb_hbm, out_hbm,
               a_idx, b_idx, atom2, comb2, si0, si1, sg0, sg1, sw0, sw1):
    wid = _wid()
    sems_i = (si0, si1)
    sems_g = (sg0, sg1)
    sems_w = (sw0, sw1)

    def idx_load(s, base, sem):
        pltpu.async_copy(a_hbm.at[pl.ds(base, CHUNK)], a_idx.at[s], sem)
        pltpu.async_copy(batch_hbm.at[pl.ds(base, CHUNK)], b_idx.at[s], sem)

    def idx_wait(s, base, sem):
        pltpu.make_async_copy(a_hbm.at[pl.ds(base, CHUNK)], a_idx.at[s],
                              sem).wait()
        pltpu.make_async_copy(batch_hbm.at[pl.ds(base, CHUNK)], b_idx.at[s],
                              sem).wait()

    def out_slices(base):
        return (out_hbm.at[pl.ds(base, CHUNK), pl.ds(0, D_NODE)],
                out_hbm.at[pl.ds(base, CHUNK), pl.ds(D_NODE, 2 * D_NODE)])

    for s in (0, 1):
        idx_load(s, (wid + s * NW) * CHUNK, sems_i[s])

    def pair_body(i, _):
        for s in (0, 1):
            k = 2 * i + s
            c = wid + k * NW
            base = c * CHUNK

            @pl.when(c < N_FULL_NODE_CHUNKS)
            def _(k=k, c=c, base=base, s=s):
                o_a, o_c = out_slices(base)
                @pl.when(k >= 2)
                def _():
                    pltpu.make_async_copy(atom2.at[s], o_a, sems_w[s]).wait()
                    pltpu.make_async_copy(comb2.at[s], o_c, sems_w[s]).wait()

                idx_wait(s, base, sems_i[s])
                ga = pltpu.async_copy(atom_hbm.at[a_idx.at[s]], atom2.at[s],
                                      sems_g[s])
                gc = pltpu.async_copy(comb_hbm.at[b_idx.at[s]], comb2.at[s],
                                      sems_g[s])
                ga.wait()
                gc.wait()

                @pl.when(c + 2 * NW < N_FULL_NODE_CHUNKS)
                def _():
                    idx_load(s, (c + 2 * NW) * CHUNK, sems_i[s])

                pltpu.async_copy(atom2.at[s], o_a, sems_w[s])
                pltpu.async_copy(comb2.at[s], o_c, sems_w[s])
        return 0

    max_pairs = (N_FULL_NODE_CHUNKS // NW + 2) // 2
    lax.fori_loop(0, max_pairs, pair_body, 0)

    for s in (0, 1):
        o_a, o_c = out_slices(wid * CHUNK)
        pltpu.make_async_copy(atom2.at[s], o_a, sems_w[s]).wait()
        pltpu.make_async_copy(comb2.at[s], o_c, sems_w[s]).wait()

    @pl.when(wid == NW - 1)
    def _tail():
        base = N_FULL_NODE_CHUNKS * CHUNK
        pltpu.sync_copy(a_hbm.at[pl.ds(base, NODE_TAIL)],
                        a_idx.at[0, pl.ds(0, NODE_TAIL)])
        pltpu.sync_copy(batch_hbm.at[pl.ds(base, NODE_TAIL)],
                        b_idx.at[0, pl.ds(0, NODE_TAIL)])
        ca = pltpu.async_copy(atom_hbm.at[a_idx.at[0, pl.ds(0, NODE_TAIL)]],
                              atom2.at[0, pl.ds(0, NODE_TAIL)], sg0)
        cb = pltpu.async_copy(comb_hbm.at[b_idx.at[0, pl.ds(0, NODE_TAIL)]],
                              comb2.at[0, pl.ds(0, NODE_TAIL)], sg1)
        ca.wait()
        cb.wait()
        pltpu.sync_copy(
            atom2.at[0, pl.ds(0, NODE_TAIL)],
            out_hbm.at[pl.ds(base, NODE_TAIL), pl.ds(0, D_NODE)])
        pltpu.sync_copy(
            comb2.at[0, pl.ds(0, NODE_TAIL)],
            out_hbm.at[pl.ds(base, NODE_TAIL), pl.ds(D_NODE, 2 * D_NODE)])


EDGES_PER_TILE = N_EDGES // NW
E_GROUP = 8 * CHUNK
E_NGROUPS = EDGES_PER_TILE // E_GROUP
E_TAIL = EDGES_PER_TILE - E_NGROUPS * E_GROUP
E_TAIL_FULL = E_TAIL // CHUNK
E_TAIL_REM = E_TAIL - E_TAIL_FULL * CHUNK


@functools.partial(
    pl.kernel,
    out_type=jax.ShapeDtypeStruct((N_EDGES * D_EDGE,), jnp.float32),
    mesh=_mesh,
    compiler_params=_params,
    scratch_types=[
        pltpu.VMEM((EDGE_VOCAB * D_EDGE,), jnp.float32),
        pltpu.VMEM((2, E_GROUP), jnp.int32),
        pltpu.VMEM((2, E_GROUP * D_EDGE), jnp.float32),
        pltpu.SemaphoreType.DMA,
        pltpu.SemaphoreType.DMA,
        pltpu.SemaphoreType.DMA,
        pltpu.SemaphoreType.DMA,
    ],
)
def _edge_kernel(e_hbm, et_hbm, out_hbm, table_v, idx2, rows2,
                 si0, si1, sw0, sw1):
    wid = _wid()
    e0 = wid * EDGES_PER_TILE
    pltpu.sync_copy(et_hbm, table_v)
    sems_i = (si0, si1)
    sems_w = (sw0, sw1)
    lane = lax.iota(jnp.int32, 16)
    lane32 = lane * D_EDGE

    def build_rows(s, n_vecs):
        def body(j, _):
            e_vec = idx2[s, pl.ds(j * 16, 16)]
            src0 = e_vec * D_EDGE
            dst0 = j * (16 * D_EDGE) + lane32
            for col in range(D_EDGE):
                vals = plsc.load_gather(table_v, [src0 + col])
                plsc.store_scatter(rows2.at[s], [dst0 + col], vals)
            return 0

        lax.fori_loop(0, n_vecs, body, 0)

    for s in (0, 1):
        pltpu.async_copy(e_hbm.at[pl.ds(e0 + s * E_GROUP, E_GROUP)],
                         idx2.at[s], sems_i[s])

    def pair_body(i, _):
        for s in (0, 1):
            g = 2 * i + s
            base = (e0 + g * E_GROUP) * D_EDGE

            @pl.when(g >= 2)
            def _(s=s, base=base):
                pltpu.make_async_copy(
                    rows2.at[s], out_hbm.at[pl.ds(base, E_GROUP * D_EDGE)],
                    sems_w[s]).wait()

            pltpu.make_async_copy(e_hbm.at[pl.ds(e0 + g * E_GROUP, E_GROUP)],
                                  idx2.at[s], sems_i[s]).wait()
            build_rows(s, E_GROUP // 16)

            @pl.when(g + 2 < E_NGROUPS)
            def _(s=s, g=g):
                pltpu.async_copy(
                    e_hbm.at[pl.ds(e0 + (g + 2) * E_GROUP, E_GROUP)],
                    idx2.at[s], sems_i[s])

            pltpu.async_copy(rows2.at[s],
                             out_hbm.at[pl.ds(base, E_GROUP * D_EDGE)],
                             sems_w[s])
        return 0

    lax.fori_loop(0, E_NGROUPS // 2, pair_body, 0)

    for s in (0, 1):
        pltpu.make_async_copy(rows2.at[s],
                              out_hbm.at[pl.ds(e0 * D_EDGE,
                                               E_GROUP * D_EDGE)],
                              sems_w[s]).wait()

    tbase = e0 + E_NGROUPS * E_GROUP
    pltpu.sync_copy(e_hbm.at[pl.ds(tbase, E_TAIL)],
                    idx2.at[0, pl.ds(0, E_TAIL)])
    build_rows(0, E_TAIL // 16)
    pltpu.sync_copy(rows2.at[0, pl.ds(0, E_TAIL * D_EDGE)],
                    out_hbm.at[pl.ds(tbase * D_EDGE, E_TAIL * D_EDGE)])


def kernel(a, c, e, edge_index, t, batch, atom_table, node_count_table,
           time_table, edge_table):
    del c
    a = a.astype(jnp.int32)
    e = e.astype(jnp.int32)
    t = t.astype(jnp.int32)
    batch = batch.astype(jnp.int32)
    combined = _combined_kernel(batch, t, node_count_table, time_table)
    h_0 = _h0_kernel(a, batch, atom_table, combined)
    e_embed = _edge_kernel(e, edge_table.reshape(-1))
    e_embed = e_embed.reshape(N_EDGES, D_EDGE)
    return (h_0, (edge_index[0], edge_index[1]), e_embed)


# speedup vs baseline: 3.6979x; 1.1889x over previous
import functools

import jax
import jax.numpy as jnp
from jax import lax
from jax.experimental import pallas as pl
from jax.experimental.pallas import tpu as pltpu
from jax.experimental.pallas import tpu_sc as plsc

N_NODES = 100000
N_EDGES = 1600000
NUM_GRAPHS = 2048
ATOM_VOCAB = 100
EDGE_VOCAB = 8
TIME_STEPS = 1000
MAX_NODE_COUNT = 512
D_NODE = 128
D_EDGE = 32

NC = 2
NS = 16
NW = NC * NS
G_PER_W = NUM_GRAPHS // NW

CHUNK = 128
N_FULL_NODE_CHUNKS = N_NODES // CHUNK
NODE_TAIL = N_NODES - N_FULL_NODE_CHUNKS * CHUNK
N_EDGE_CHUNKS = N_EDGES // CHUNK

_mesh = plsc.VectorSubcoreMesh(core_axis_name="c", subcore_axis_name="s")
_params = pltpu.CompilerParams(needs_layout_passes=False,
                               use_tc_tiling_on_sc=False)


def _wid():
    return lax.axis_index("s") * NC + lax.axis_index("c")


@functools.partial(
    pl.kernel,
    out_type=jax.ShapeDtypeStruct((NUM_GRAPHS, 2 * D_NODE), jnp.float32),
    mesh=_mesh,
    compiler_params=_params,
    scratch_types=[
        pltpu.VMEM((N_NODES,), jnp.int32),
        pltpu.VMEM((G_PER_W + 16,), jnp.int32),
        pltpu.VMEM((G_PER_W,), jnp.int32),
        pltpu.VMEM((G_PER_W,), jnp.int32),
        pltpu.VMEM((G_PER_W, D_NODE), jnp.float32),
        pltpu.VMEM((G_PER_W, D_NODE), jnp.float32),
        pltpu.SemaphoreType.DMA,
    ],
)
def _combined_kernel(batch_hbm, t_hbm, nct_hbm, tt_hbm, out_hbm,
                     batch_v, starts_v, nc_idx_v, t_idx_v, nc_rows_v,
                     t_rows_v, sem):
    wid = _wid()
    g0 = wid * G_PER_W
    pltpu.sync_copy(batch_hbm, batch_v)
    lane = lax.iota(jnp.int32, 16)
    for j in range(G_PER_W // 16 + 1):
        g_vec = g0 + j * 16 + lane

        def body(_, carry, g_vec=g_vec):
            lo, hi = carry
            active = lo < hi
            mid = lax.shift_right_logical(lo + hi, 1)
            vals = plsc.load_gather(
                batch_v, [jnp.minimum(mid, N_NODES - 1)])
            go = jnp.logical_and(active, vals < g_vec)
            lo = jnp.where(go, mid + 1, lo)
            hi = jnp.where(jnp.logical_and(active, jnp.logical_not(go)),
                           mid, hi)
            return lo, hi

        lo = jnp.zeros((16,), jnp.int32)
        hi = jnp.full((16,), N_NODES, jnp.int32)
        lo, hi = lax.fori_loop(0, 17, body, (lo, hi))
        starts_v[pl.ds(j * 16, 16)] = lo
    for j in range(G_PER_W // 16):
        s0 = starts_v[pl.ds(j * 16, 16)]
        s1 = starts_v[pl.ds(j * 16 + 1, 16)]
        n = s1 - s0
        nc_idx_v[pl.ds(j * 16, 16)] = jnp.minimum(n, MAX_NODE_COUNT - 1)
    pltpu.sync_copy(t_hbm.at[pl.ds(g0, G_PER_W)], t_idx_v)
    pltpu.async_copy(nct_hbm.at[nc_idx_v], nc_rows_v, sem).wait()
    pltpu.async_copy(tt_hbm.at[t_idx_v], t_rows_v, sem).wait()
    pltpu.sync_copy(nc_rows_v,
                    out_hbm.at[pl.ds(g0, G_PER_W), pl.ds(0, D_NODE)])
    pltpu.sync_copy(t_rows_v,
                    out_hbm.at[pl.ds(g0, G_PER_W), pl.ds(D_NODE, D_NODE)])


@functools.partial(
    pl.kernel,
    out_type=jax.ShapeDtypeStruct((N_NODES, 3 * D_NODE), jnp.float32),
    mesh=_mesh,
    compiler_params=_params,
    scratch_types=[
        pltpu.VMEM((2, CHUNK), jnp.int32),
        pltpu.VMEM((2, CHUNK), jnp.int32),
        pltpu.VMEM((2, CHUNK, D_NODE), jnp.float32),
        pltpu.VMEM((2, CHUNK, 2 * D_NODE), jnp.float32),
        pltpu.SemaphoreType.DMA,
        pltpu.SemaphoreType.DMA,
        pltpu.SemaphoreType.DMA,
        pltpu.SemaphoreType.DMA,
        pltpu.SemaphoreType.DMA,
        pltpu.SemaphoreType.DMA,
    ],
)
def _h0_kernel(a_hbm, batch_hbm, atom_hbm, comb_hbm, out_hbm,
               a_idx, b_idx, atom2, comb2, si0, si1, sg0, sg1, sw0, sw1):
    wid = _wid()
    sems_i = (si0, si1)
    sems_g = (sg0, sg1)
    sems_w = (sw0, sw1)

    def idx_load(s, base, sem):
        pltpu.async_copy(a_hbm.at[pl.ds(base, CHUNK)], a_idx.at[s], sem)
        pltpu.async_copy(batch_hbm.at[pl.ds(base, CHUNK)], b_idx.at[s], sem)

    def idx_wait(s, base, sem):
        pltpu.make_async_copy(a_hbm.at[pl.ds(base, CHUNK)], a_idx.at[s],
                              sem).wait()
        pltpu.make_async_copy(batch_hbm.at[pl.ds(base, CHUNK)], b_idx.at[s],
                              sem).wait()

    def out_slices(base):
        return (out_hbm.at[pl.ds(base, CHUNK), pl.ds(0, D_NODE)],
                out_hbm.at[pl.ds(base, CHUNK), pl.ds(D_NODE, 2 * D_NODE)])

    for s in (0, 1):
        idx_load(s, (wid + s * NW) * CHUNK, sems_i[s])

    def pair_body(i, _):
        for s in (0, 1):
            k = 2 * i + s
            c = wid + k * NW
            base = c * CHUNK

            @pl.when(c < N_FULL_NODE_CHUNKS)
            def _(k=k, c=c, base=base, s=s):
                o_a, o_c = out_slices(base)
                @pl.when(k >= 2)
                def _():
                    pltpu.make_async_copy(atom2.at[s], o_a, sems_w[s]).wait()
                    pltpu.make_async_copy(comb2.at[s], o_c, sems_w[s]).wait()

                idx_wait(s, base, sems_i[s])
                ga = pltpu.async_copy(atom_hbm.at[a_idx.at[s]], atom2.at[s],
                                      sems_g[s])
                gc = pltpu.async_copy(comb_hbm.at[b_idx.at[s]], comb2.at[s],
                                      sems_g[s])
                ga.wait()
                gc.wait()

                @pl.when(c + 2 * NW < N_FULL_NODE_CHUNKS)
                def _():
                    idx_load(s, (c + 2 * NW) * CHUNK, sems_i[s])

                pltpu.async_copy(atom2.at[s], o_a, sems_w[s])
                pltpu.async_copy(comb2.at[s], o_c, sems_w[s])
        return 0

    max_pairs = (N_FULL_NODE_CHUNKS // NW + 2) // 2
    lax.fori_loop(0, max_pairs, pair_body, 0)

    for s in (0, 1):
        o_a, o_c = out_slices(wid * CHUNK)
        pltpu.make_async_copy(atom2.at[s], o_a, sems_w[s]).wait()
        pltpu.make_async_copy(comb2.at[s], o_c, sems_w[s]).wait()

    @pl.when(wid == NW - 1)
    def _tail():
        base = N_FULL_NODE_CHUNKS * CHUNK
        pltpu.sync_copy(a_hbm.at[pl.ds(base, NODE_TAIL)],
                        a_idx.at[0, pl.ds(0, NODE_TAIL)])
        pltpu.sync_copy(batch_hbm.at[pl.ds(base, NODE_TAIL)],
                        b_idx.at[0, pl.ds(0, NODE_TAIL)])
        ca = pltpu.async_copy(atom_hbm.at[a_idx.at[0, pl.ds(0, NODE_TAIL)]],
                              atom2.at[0, pl.ds(0, NODE_TAIL)], sg0)
        cb = pltpu.async_copy(comb_hbm.at[b_idx.at[0, pl.ds(0, NODE_TAIL)]],
                              comb2.at[0, pl.ds(0, NODE_TAIL)], sg1)
        ca.wait()
        cb.wait()
        pltpu.sync_copy(
            atom2.at[0, pl.ds(0, NODE_TAIL)],
            out_hbm.at[pl.ds(base, NODE_TAIL), pl.ds(0, D_NODE)])
        pltpu.sync_copy(
            comb2.at[0, pl.ds(0, NODE_TAIL)],
            out_hbm.at[pl.ds(base, NODE_TAIL), pl.ds(D_NODE, 2 * D_NODE)])


EDGES_PER_TILE = N_EDGES // NW
E_GROUP = 8 * CHUNK
E_NGROUPS = EDGES_PER_TILE // E_GROUP
E_TAIL = EDGES_PER_TILE - E_NGROUPS * E_GROUP
E_TAIL_FULL = E_TAIL // CHUNK
E_TAIL_REM = E_TAIL - E_TAIL_FULL * CHUNK


@functools.partial(
    pl.kernel,
    out_type=jax.ShapeDtypeStruct((N_EDGES * D_EDGE,), jnp.float32),
    mesh=_mesh,
    compiler_params=_params,
    scratch_types=[
        pltpu.VMEM((EDGE_VOCAB * D_EDGE,), jnp.float32),
        pltpu.VMEM((2, E_GROUP), jnp.int32),
        pltpu.VMEM((2, E_GROUP * D_EDGE), jnp.float32),
        pltpu.SemaphoreType.DMA,
        pltpu.SemaphoreType.DMA,
        pltpu.SemaphoreType.DMA,
        pltpu.SemaphoreType.DMA,
    ],
)
def _edge_kernel(e_hbm, et_hbm, out_hbm, table_v, idx2, rows2,
                 si0, si1, sw0, sw1):
    wid = _wid()
    e0 = wid * EDGES_PER_TILE
    pltpu.sync_copy(et_hbm, table_v)
    sems_i = (si0, si1)
    sems_w = (sw0, sw1)
    lane = lax.iota(jnp.int32, 16)
    lane32 = lane * D_EDGE

    def build_rows(s, n_vecs):
        @plsc.parallel_loop(0, n_vecs, step=1, unroll=2)
        def body(j):
            e_vec = idx2[s, pl.ds(j * 16, 16)]
            src0 = e_vec * D_EDGE
            dst0 = j * (16 * D_EDGE) + lane32
            vals = [plsc.load_gather(table_v, [src0 + col])
                    for col in range(D_EDGE)]
            for col in range(D_EDGE):
                plsc.store_scatter(rows2.at[s], [dst0 + col], vals[col])

    for s in (0, 1):
        pltpu.async_copy(e_hbm.at[pl.ds(e0 + s * E_GROUP, E_GROUP)],
                         idx2.at[s], sems_i[s])

    def pair_body(i, _):
        for s in (0, 1):
            g = 2 * i + s
            base = (e0 + g * E_GROUP) * D_EDGE

            @pl.when(g >= 2)
            def _(s=s, base=base):
                pltpu.make_async_copy(
                    rows2.at[s], out_hbm.at[pl.ds(base, E_GROUP * D_EDGE)],
                    sems_w[s]).wait()

            pltpu.make_async_copy(e_hbm.at[pl.ds(e0 + g * E_GROUP, E_GROUP)],
                                  idx2.at[s], sems_i[s]).wait()
            build_rows(s, E_GROUP // 16)

            @pl.when(g + 2 < E_NGROUPS)
            def _(s=s, g=g):
                pltpu.async_copy(
                    e_hbm.at[pl.ds(e0 + (g + 2) * E_GROUP, E_GROUP)],
                    idx2.at[s], sems_i[s])

            pltpu.async_copy(rows2.at[s],
                             out_hbm.at[pl.ds(base, E_GROUP * D_EDGE)],
                             sems_w[s])
        return 0

    lax.fori_loop(0, E_NGROUPS // 2, pair_body, 0)

    for s in (0, 1):
        pltpu.make_async_copy(rows2.at[s],
                              out_hbm.at[pl.ds(e0 * D_EDGE,
                                               E_GROUP * D_EDGE)],
                              sems_w[s]).wait()

    tbase = e0 + E_NGROUPS * E_GROUP
    pltpu.sync_copy(e_hbm.at[pl.ds(tbase, E_TAIL)],
                    idx2.at[0, pl.ds(0, E_TAIL)])
    build_rows(0, E_TAIL // 16)
    pltpu.sync_copy(rows2.at[0, pl.ds(0, E_TAIL * D_EDGE)],
                    out_hbm.at[pl.ds(tbase * D_EDGE, E_TAIL * D_EDGE)])


def kernel(a, c, e, edge_index, t, batch, atom_table, node_count_table,
           time_table, edge_table):
    del c
    a = a.astype(jnp.int32)
    e = e.astype(jnp.int32)
    t = t.astype(jnp.int32)
    batch = batch.astype(jnp.int32)
    combined = _combined_kernel(batch, t, node_count_table, time_table)
    h_0 = _h0_kernel(a, batch, atom_table, combined)
    e_embed = _edge_kernel(e, edge_table.reshape(-1))
    e_embed = e_embed.reshape(N_EDGES, D_EDGE)
    return (h_0, (edge_index[0], edge_index[1]), e_embed)


# speedup vs baseline: 5.0602x; 1.3684x over previous
import functools

import jax
import jax.numpy as jnp
from jax import lax
from jax.experimental import pallas as pl
from jax.experimental.pallas import tpu as pltpu
from jax.experimental.pallas import tpu_sc as plsc

N_NODES = 100000
N_EDGES = 1600000
NUM_GRAPHS = 2048
ATOM_VOCAB = 100
EDGE_VOCAB = 8
TIME_STEPS = 1000
MAX_NODE_COUNT = 512
D_NODE = 128
D_EDGE = 32

NC = 2
NS = 16
NW = NC * NS
G_PER_W = NUM_GRAPHS // NW

CHUNK = 128
N_FULL_NODE_CHUNKS = N_NODES // CHUNK
NODE_TAIL = N_NODES - N_FULL_NODE_CHUNKS * CHUNK
N_EDGE_CHUNKS = N_EDGES // CHUNK

_mesh = plsc.VectorSubcoreMesh(core_axis_name="c", subcore_axis_name="s")
_params = pltpu.CompilerParams(needs_layout_passes=False,
                               use_tc_tiling_on_sc=False)


def _wid():
    return lax.axis_index("s") * NC + lax.axis_index("c")


@functools.partial(
    pl.kernel,
    out_type=jax.ShapeDtypeStruct((NUM_GRAPHS, 2 * D_NODE), jnp.float32),
    mesh=_mesh,
    compiler_params=_params,
    scratch_types=[
        pltpu.VMEM((N_NODES,), jnp.int32),
        pltpu.VMEM((G_PER_W + 16,), jnp.int32),
        pltpu.VMEM((G_PER_W,), jnp.int32),
        pltpu.VMEM((G_PER_W,), jnp.int32),
        pltpu.VMEM((G_PER_W, D_NODE), jnp.float32),
        pltpu.VMEM((G_PER_W, D_NODE), jnp.float32),
        pltpu.SemaphoreType.DMA,
    ],
)
def _combined_kernel(batch_hbm, t_hbm, nct_hbm, tt_hbm, out_hbm,
                     batch_v, starts_v, nc_idx_v, t_idx_v, nc_rows_v,
                     t_rows_v, sem):
    wid = _wid()
    g0 = wid * G_PER_W
    pltpu.sync_copy(batch_hbm, batch_v)
    lane = lax.iota(jnp.int32, 16)
    for j in range(G_PER_W // 16 + 1):
        g_vec = g0 + j * 16 + lane

        def body(_, carry, g_vec=g_vec):
            lo, hi = carry
            active = lo < hi
            mid = lax.shift_right_logical(lo + hi, 1)
            vals = plsc.load_gather(
                batch_v, [jnp.minimum(mid, N_NODES - 1)])
            go = jnp.logical_and(active, vals < g_vec)
            lo = jnp.where(go, mid + 1, lo)
            hi = jnp.where(jnp.logical_and(active, jnp.logical_not(go)),
                           mid, hi)
            return lo, hi

        lo = jnp.zeros((16,), jnp.int32)
        hi = jnp.full((16,), N_NODES, jnp.int32)
        lo, hi = lax.fori_loop(0, 17, body, (lo, hi))
        starts_v[pl.ds(j * 16, 16)] = lo
    for j in range(G_PER_W // 16):
        s0 = starts_v[pl.ds(j * 16, 16)]
        s1 = starts_v[pl.ds(j * 16 + 1, 16)]
        n = s1 - s0
        nc_idx_v[pl.ds(j * 16, 16)] = jnp.minimum(n, MAX_NODE_COUNT - 1)
    pltpu.sync_copy(t_hbm.at[pl.ds(g0, G_PER_W)], t_idx_v)
    pltpu.async_copy(nct_hbm.at[nc_idx_v], nc_rows_v, sem).wait()
    pltpu.async_copy(tt_hbm.at[t_idx_v], t_rows_v, sem).wait()
    pltpu.sync_copy(nc_rows_v,
                    out_hbm.at[pl.ds(g0, G_PER_W), pl.ds(0, D_NODE)])
    pltpu.sync_copy(t_rows_v,
                    out_hbm.at[pl.ds(g0, G_PER_W), pl.ds(D_NODE, D_NODE)])


@functools.partial(
    pl.kernel,
    out_type=jax.ShapeDtypeStruct((N_NODES, 3 * D_NODE), jnp.float32),
    mesh=_mesh,
    compiler_params=_params,
    scratch_types=[
        pltpu.VMEM((2, CHUNK), jnp.int32),
        pltpu.VMEM((2, CHUNK), jnp.int32),
        pltpu.VMEM((2, CHUNK, D_NODE), jnp.float32),
        pltpu.VMEM((2, CHUNK, 2 * D_NODE), jnp.float32),
        pltpu.SemaphoreType.DMA,
        pltpu.SemaphoreType.DMA,
        pltpu.SemaphoreType.DMA,
        pltpu.SemaphoreType.DMA,
        pltpu.SemaphoreType.DMA,
        pltpu.SemaphoreType.DMA,
    ],
)
def _h0_kernel(a_hbm, batch_hbm, atom_hbm, comb_hbm, out_hbm,
               a_idx, b_idx, atom2, comb2, si0, si1, sg0, sg1, sw0, sw1):
    wid = _wid()
    sems_i = (si0, si1)
    sems_g = (sg0, sg1)
    sems_w = (sw0, sw1)

    def idx_load(s, base, sem):
        pltpu.async_copy(a_hbm.at[pl.ds(base, CHUNK)], a_idx.at[s], sem)
        pltpu.async_copy(batch_hbm.at[pl.ds(base, CHUNK)], b_idx.at[s], sem)

    def idx_wait(s, base, sem):
        pltpu.make_async_copy(a_hbm.at[pl.ds(base, CHUNK)], a_idx.at[s],
                              sem).wait()
        pltpu.make_async_copy(batch_hbm.at[pl.ds(base, CHUNK)], b_idx.at[s],
                              sem).wait()

    def out_slices(base):
        return (out_hbm.at[pl.ds(base, CHUNK), pl.ds(0, D_NODE)],
                out_hbm.at[pl.ds(base, CHUNK), pl.ds(D_NODE, 2 * D_NODE)])

    for s in (0, 1):
        idx_load(s, (wid + s * NW) * CHUNK, sems_i[s])

    def pair_body(i, _):
        for s in (0, 1):
            k = 2 * i + s
            c = wid + k * NW
            base = c * CHUNK

            @pl.when(c < N_FULL_NODE_CHUNKS)
            def _(k=k, c=c, base=base, s=s):
                o_a, o_c = out_slices(base)
                @pl.when(k >= 2)
                def _():
                    pltpu.make_async_copy(atom2.at[s], o_a, sems_w[s]).wait()
                    pltpu.make_async_copy(comb2.at[s], o_c, sems_w[s]).wait()

                idx_wait(s, base, sems_i[s])
                ga = pltpu.async_copy(atom_hbm.at[a_idx.at[s]], atom2.at[s],
                                      sems_g[s])
                gc = pltpu.async_copy(comb_hbm.at[b_idx.at[s]], comb2.at[s],
                                      sems_g[s])
                ga.wait()
                gc.wait()

                @pl.when(c + 2 * NW < N_FULL_NODE_CHUNKS)
                def _():
                    idx_load(s, (c + 2 * NW) * CHUNK, sems_i[s])

                pltpu.async_copy(atom2.at[s], o_a, sems_w[s])
                pltpu.async_copy(comb2.at[s], o_c, sems_w[s])
        return 0

    max_pairs = (N_FULL_NODE_CHUNKS // NW + 2) // 2
    lax.fori_loop(0, max_pairs, pair_body, 0)

    for s in (0, 1):
        o_a, o_c = out_slices(wid * CHUNK)
        pltpu.make_async_copy(atom2.at[s], o_a, sems_w[s]).wait()
        pltpu.make_async_copy(comb2.at[s], o_c, sems_w[s]).wait()

    @pl.when(wid == NW - 1)
    def _tail():
        base = N_FULL_NODE_CHUNKS * CHUNK
        pltpu.sync_copy(a_hbm.at[pl.ds(base, NODE_TAIL)],
                        a_idx.at[0, pl.ds(0, NODE_TAIL)])
        pltpu.sync_copy(batch_hbm.at[pl.ds(base, NODE_TAIL)],
                        b_idx.at[0, pl.ds(0, NODE_TAIL)])
        ca = pltpu.async_copy(atom_hbm.at[a_idx.at[0, pl.ds(0, NODE_TAIL)]],
                              atom2.at[0, pl.ds(0, NODE_TAIL)], sg0)
        cb = pltpu.async_copy(comb_hbm.at[b_idx.at[0, pl.ds(0, NODE_TAIL)]],
                              comb2.at[0, pl.ds(0, NODE_TAIL)], sg1)
        ca.wait()
        cb.wait()
        pltpu.sync_copy(
            atom2.at[0, pl.ds(0, NODE_TAIL)],
            out_hbm.at[pl.ds(base, NODE_TAIL), pl.ds(0, D_NODE)])
        pltpu.sync_copy(
            comb2.at[0, pl.ds(0, NODE_TAIL)],
            out_hbm.at[pl.ds(base, NODE_TAIL), pl.ds(D_NODE, 2 * D_NODE)])


EDGES_PER_TILE = N_EDGES // NW
E_GROUP = 8 * CHUNK
E_NGROUPS = EDGES_PER_TILE // E_GROUP
E_TAIL = EDGES_PER_TILE - E_NGROUPS * E_GROUP
E_TAIL_FULL = E_TAIL // CHUNK
E_TAIL_REM = E_TAIL - E_TAIL_FULL * CHUNK


@functools.partial(
    pl.kernel,
    out_type=jax.ShapeDtypeStruct((N_EDGES * D_EDGE,), jnp.float32),
    mesh=_mesh,
    compiler_params=_params,
    scratch_types=[
        pltpu.VMEM((EDGE_VOCAB * D_EDGE,), jnp.float32),
        pltpu.VMEM((2, E_GROUP), jnp.int32),
        pltpu.VMEM((2, E_GROUP * D_EDGE), jnp.float32),
        pltpu.SemaphoreType.DMA,
        pltpu.SemaphoreType.DMA,
        pltpu.SemaphoreType.DMA,
        pltpu.SemaphoreType.DMA,
    ],
)
def _edge_kernel(e_hbm, et_hbm, out_hbm, table_v, idx2, rows2,
                 si0, si1, sw0, sw1):
    wid = _wid()
    e0 = wid * EDGES_PER_TILE
    pltpu.sync_copy(et_hbm, table_v)
    sems_i = (si0, si1)
    sems_w = (sw0, sw1)
    lane = lax.iota(jnp.int32, 16)
    lane32 = lane * D_EDGE

    def build_rows(s, n_vecs):
        depth = 6

        @plsc.parallel_loop(0, n_vecs, step=1)
        def body(j):
            e_vec = idx2[s, pl.ds(j * 16, 16)]
            src0 = e_vec * D_EDGE
            dst0 = j * (16 * D_EDGE) + lane32
            vals = {}
            for col in range(D_EDGE + depth):
                if col < D_EDGE:
                    vals[col] = plsc.load_gather(table_v, [src0 + col])
                if col >= depth:
                    plsc.store_scatter(rows2.at[s], [dst0 + (col - depth)],
                                       vals.pop(col - depth))

    for s in (0, 1):
        pltpu.async_copy(e_hbm.at[pl.ds(e0 + s * E_GROUP, E_GROUP)],
                         idx2.at[s], sems_i[s])

    def pair_body(i, _):
        for s in (0, 1):
            g = 2 * i + s
            base = (e0 + g * E_GROUP) * D_EDGE

            @pl.when(g >= 2)
            def _(s=s, base=base):
                pltpu.make_async_copy(
                    rows2.at[s], out_hbm.at[pl.ds(base, E_GROUP * D_EDGE)],
                    sems_w[s]).wait()

            pltpu.make_async_copy(e_hbm.at[pl.ds(e0 + g * E_GROUP, E_GROUP)],
                                  idx2.at[s], sems_i[s]).wait()
            build_rows(s, E_GROUP // 16)

            @pl.when(g + 2 < E_NGROUPS)
            def _(s=s, g=g):
                pltpu.async_copy(
                    e_hbm.at[pl.ds(e0 + (g + 2) * E_GROUP, E_GROUP)],
                    idx2.at[s], sems_i[s])

            pltpu.async_copy(rows2.at[s],
                             out_hbm.at[pl.ds(base, E_GROUP * D_EDGE)],
                             sems_w[s])
        return 0

    lax.fori_loop(0, E_NGROUPS // 2, pair_body, 0)

    for s in (0, 1):
        pltpu.make_async_copy(rows2.at[s],
                              out_hbm.at[pl.ds(e0 * D_EDGE,
                                               E_GROUP * D_EDGE)],
                              sems_w[s]).wait()

    tbase = e0 + E_NGROUPS * E_GROUP
    pltpu.sync_copy(e_hbm.at[pl.ds(tbase, E_TAIL)],
                    idx2.at[0, pl.ds(0, E_TAIL)])
    build_rows(0, E_TAIL // 16)
    pltpu.sync_copy(rows2.at[0, pl.ds(0, E_TAIL * D_EDGE)],
                    out_hbm.at[pl.ds(tbase * D_EDGE, E_TAIL * D_EDGE)])


def kernel(a, c, e, edge_index, t, batch, atom_table, node_count_table,
           time_table, edge_table):
    del c
    a = a.astype(jnp.int32)
    e = e.astype(jnp.int32)
    t = t.astype(jnp.int32)
    batch = batch.astype(jnp.int32)
    combined = _combined_kernel(batch, t, node_count_table, time_table)
    h_0 = _h0_kernel(a, batch, atom_table, combined)
    e_embed = _edge_kernel(e, edge_table.reshape(-1))
    e_embed = e_embed.reshape(N_EDGES, D_EDGE)
    return (h_0, (edge_index[0], edge_index[1]), e_embed)
